# baseline (speedup 1.0000x reference)
import functools

import jax
import jax.numpy as jnp
from jax import lax
from jax.experimental import pallas as pl
from jax.experimental.pallas import tpu as pltpu
from jax.experimental.pallas import tpu_sc as plsc

EMBED = 2048
NE = 16
NTOK = 16384
ACTIVE = 2

NC, NS, L = 2, 16, 16
NW = NC * NS
TPW = NTOK // NW
GROUPS = TPW // L

ROW_BLK = 1024


def _score_body(x_ref, wt_ref, o_ref):
    o_ref[...] = lax.dot_general(wt_ref[...], x_ref[...],
                                 (((1,), (1,)), ((), ())),
                                 preferred_element_type=jnp.float32)


def _scores_tc(x, Wt):
    rows = x.shape[0]
    return pl.pallas_call(
        _score_body,
        grid=(rows // ROW_BLK,),
        in_specs=[
            pl.BlockSpec((ROW_BLK, EMBED), lambda i: (i, 0)),
            pl.BlockSpec((NE, EMBED), lambda i: (0, 0)),
        ],
        out_specs=pl.BlockSpec((NE, ROW_BLK), lambda i: (0, i)),
        out_shape=jax.ShapeDtypeStruct((NE, rows), jnp.float32),
    )(x, Wt)


def _transpose_body(rw_ref, mk_ref, orw_ref, omk_ref):
    orw_ref[...] = rw_ref[...].T
    omk_ref[...] = mk_ref[...].T


def _transpose_tc(rw_t, mk_t):
    return pl.pallas_call(
        _transpose_body,
        grid=(NTOK // ROW_BLK,),
        in_specs=[
            pl.BlockSpec((NE, ROW_BLK), lambda i: (0, i)),
            pl.BlockSpec((NE, ROW_BLK), lambda i: (0, i)),
        ],
        out_specs=[
            pl.BlockSpec((ROW_BLK, NE), lambda i: (i, 0)),
            pl.BlockSpec((ROW_BLK, NE), lambda i: (i, 0)),
        ],
        out_shape=[
            jax.ShapeDtypeStruct((NTOK, NE), jnp.float32),
            jax.ShapeDtypeStruct((NTOK, NE), jnp.float32),
        ],
    )(rw_t, mk_t)


TCHUNK = 128
CGROUPS = TCHUNK // L


def _route_body(tpw, scores_hbm, b_hbm, rw_hbm, mk_hbm, s_v, b_v, rw_v, mk_v):
    wid = lax.axis_index("s") * NC + lax.axis_index("c")
    base = wid * tpw
    pltpu.sync_copy(scores_hbm.at[:, pl.ds(base, tpw)], s_v)
    pltpu.sync_copy(b_hbm, b_v)
    b_vec = b_v[...]

    zeros = jnp.zeros((L,), jnp.float32)
    ones = jnp.ones((L,), jnp.float32)

    def group(g, carry):
        t0 = g * L
        toks = t0 + lax.iota(jnp.int32, L)
        m1 = jnp.full((L,), -jnp.inf, jnp.float32)
        m2 = jnp.full((L,), -jnp.inf, jnp.float32)
        i1 = jnp.zeros((L,), jnp.int32)
        i2 = jnp.zeros((L,), jnp.int32)
        for e in range(NE):
            v = jnp.maximum(s_v[e, pl.ds(t0, L)] + b_vec[e], 0.0)
            gt1 = v > m1
            gt2 = jnp.logical_and(jnp.logical_not(gt1), v > m2)
            i2 = jnp.where(gt1, i1, jnp.where(gt2, e, i2))
            m2 = jnp.where(gt1, m1, jnp.where(gt2, v, m2))
            i1 = jnp.where(gt1, e, i1)
            m1 = jnp.where(gt1, v, m1)
        t = jnp.exp(m2 - m1)
        den = 1.0 + t
        w1 = 1.0 / den
        w2 = t / den
        for e in range(NE):
            rw_v[e, pl.ds(t0, L)] = zeros
            mk_v[e, pl.ds(t0, L)] = zeros
        plsc.store_scatter(mk_v, [i1, toks], ones)
        plsc.store_scatter(mk_v, [i2, toks], ones)
        plsc.store_scatter(rw_v, [i1, toks], w1)
        plsc.store_scatter(rw_v, [i2, toks], w2)
        return carry

    lax.fori_loop(0, tpw // L, group, 0)
    pltpu.sync_copy(rw_v, rw_hbm.at[:, pl.ds(base, tpw)])
    pltpu.sync_copy(mk_v, mk_hbm.at[:, pl.ds(base, tpw)])


@functools.lru_cache(maxsize=4)
def _build_route_sc(ntok):
    tpw = ntok // NW
    return pl.kernel(
        functools.partial(_route_body, tpw),
        out_type=(
            jax.ShapeDtypeStruct((NE, ntok), jnp.float32),
            jax.ShapeDtypeStruct((NE, ntok), jnp.float32),
        ),
        mesh=plsc.VectorSubcoreMesh(core_axis_name="c", subcore_axis_name="s"),
        scratch_types=[
            pltpu.VMEM((NE, tpw), jnp.float32),
            pltpu.VMEM((NE,), jnp.float32),
            pltpu.VMEM((NE, tpw), jnp.float32),
            pltpu.VMEM((NE, tpw), jnp.float32),
        ],
        compiler_params=pltpu.CompilerParams(needs_layout_passes=False),
    )


def kernel(x, W, b):
    scores_t = _scores_tc(x, W.T)
    rw_t, mk_t = _build_route_sc(NTOK)(scores_t, b)
    return rw_t.T, mk_t.T


# speedup vs baseline: 1.0036x; 1.0036x over previous
import functools

import jax
import jax.numpy as jnp
from jax import lax
from jax.experimental import pallas as pl
from jax.experimental.pallas import tpu as pltpu
from jax.experimental.pallas import tpu_sc as plsc

EMBED = 2048
NE = 16
NTOK = 16384
ACTIVE = 2

NC, NS, L = 2, 16, 16
NW = NC * NS
TPW = NTOK // NW

ROW_BLK = 1024


def _score_body(x_ref, wt_ref, o_ref):
    o_ref[...] = lax.dot_general(wt_ref[...], x_ref[...],
                                 (((1,), (1,)), ((), ())),
                                 preferred_element_type=jnp.float32)


def _scores_tc(x, Wt):
    rows = x.shape[0]
    return pl.pallas_call(
        _score_body,
        grid=(rows // ROW_BLK,),
        in_specs=[
            pl.BlockSpec((ROW_BLK, EMBED), lambda i: (i, 0)),
            pl.BlockSpec((NE, EMBED), lambda i: (0, 0)),
        ],
        out_specs=pl.BlockSpec((NE, ROW_BLK), lambda i: (0, i)),
        out_shape=jax.ShapeDtypeStruct((NE, rows), jnp.float32),
    )(x, Wt)


def _route_body(tpw, scores_hbm, b_hbm, rw_hbm, mk_hbm, s_v, b_v, rw_v, mk_v):
    wid = lax.axis_index("s") * NC + lax.axis_index("c")
    base = wid * tpw
    pltpu.sync_copy(scores_hbm.at[:, pl.ds(base, tpw)], s_v)
    pltpu.sync_copy(b_hbm, b_v)
    b_vec = b_v[...]

    zeros = jnp.zeros((L,), jnp.float32)
    ones = jnp.ones((L,), jnp.float32)

    def group(g, carry):
        t0 = g * L
        toks = t0 + lax.iota(jnp.int32, L)
        m1 = jnp.full((L,), -jnp.inf, jnp.float32)
        m2 = jnp.full((L,), -jnp.inf, jnp.float32)
        i1 = jnp.zeros((L,), jnp.int32)
        i2 = jnp.zeros((L,), jnp.int32)
        for e in range(NE):
            v = jnp.maximum(s_v[e, pl.ds(t0, L)] + b_vec[e], 0.0)
            gt1 = v > m1
            gt2 = jnp.logical_and(jnp.logical_not(gt1), v > m2)
            i2 = jnp.where(gt1, i1, jnp.where(gt2, e, i2))
            m2 = jnp.where(gt1, m1, jnp.where(gt2, v, m2))
            i1 = jnp.where(gt1, e, i1)
            m1 = jnp.where(gt1, v, m1)
        t = jnp.exp(m2 - m1)
        den = 1.0 + t
        w1 = 1.0 / den
        w2 = t / den
        for e in range(NE):
            rw_v[e, pl.ds(t0, L)] = zeros
            mk_v[e, pl.ds(t0, L)] = zeros
        plsc.store_scatter(mk_v, [i1, toks], ones)
        plsc.store_scatter(mk_v, [i2, toks], ones)
        plsc.store_scatter(rw_v, [i1, toks], w1)
        plsc.store_scatter(rw_v, [i2, toks], w2)
        return carry

    lax.fori_loop(0, tpw // L, group, 0)
    pltpu.sync_copy(rw_v, rw_hbm.at[:, pl.ds(base, tpw)])
    pltpu.sync_copy(mk_v, mk_hbm.at[:, pl.ds(base, tpw)])


@functools.lru_cache(maxsize=4)
def _build_route_sc(ntok):
    tpw = ntok // NW
    return pl.kernel(
        functools.partial(_route_body, tpw),
        out_type=(
            jax.ShapeDtypeStruct((NE, ntok), jnp.float32),
            jax.ShapeDtypeStruct((NE, ntok), jnp.float32),
        ),
        mesh=plsc.VectorSubcoreMesh(core_axis_name="c", subcore_axis_name="s"),
        scratch_types=[
            pltpu.VMEM((NE, tpw), jnp.float32),
            pltpu.VMEM((NE,), jnp.float32),
            pltpu.VMEM((NE, tpw), jnp.float32),
            pltpu.VMEM((NE, tpw), jnp.float32),
        ],
        compiler_params=pltpu.CompilerParams(needs_layout_passes=False),
    )


def kernel(x, W, b):
    scores_t = _scores_tc(x, W.T)
    rw_t, mk_t = _build_route_sc(NTOK)(scores_t, b)
    return rw_t.T, mk_t.T
